# baseline (speedup 1.0000x reference)
import jax
import jax.numpy as jnp
from jax import lax
from jax.experimental import pallas as pl
from jax.experimental.pallas import tpu as pltpu
from jax.experimental.pallas import tpu_sc as plsc

_S = 2
_W = 8
_NW = 32


def _gather(idx, table2):
    n2 = idx.shape[0]
    d2 = table2.shape[1]
    k_per_w = n2 // _NW
    nc = k_per_w // _W
    mesh = plsc.VectorSubcoreMesh(core_axis_name="core",
                                  subcore_axis_name="subcore")

    @pl.kernel(
        out_type=jax.ShapeDtypeStruct((n2, d2), table2.dtype),
        mesh=mesh,
        scratch_types=[
            pltpu.VMEM((k_per_w,), jnp.int32),
            pltpu.VMEM((_W, d2), jnp.float32),
            pltpu.VMEM((_W, d2), jnp.float32),
            pltpu.SemaphoreType.DMA,
            pltpu.SemaphoreType.DMA,
            pltpu.SemaphoreType.DMA,
            pltpu.SemaphoreType.DMA,
            pltpu.SemaphoreType.DMA,
        ],
    )
    def k(table_hbm, i_hbm, o_hbm, idx_v, buf0, buf1, gs0, gs1, ws0, ws1,
          isem):
        cid = lax.axis_index("core")
        sid = lax.axis_index("subcore")
        wid = sid * 2 + cid
        base = wid * k_per_w
        pltpu.async_copy(i_hbm.at[pl.ds(base, k_per_w)], idx_v, isem).wait()

        @pl.loop(0, nc, step=2)
        def _(j):
            r0 = base + j * _W
            g0 = pltpu.async_copy(
                table_hbm.at[idx_v.at[pl.ds(j * _W, _W)]], buf0, gs0)
            g1 = pltpu.async_copy(
                table_hbm.at[idx_v.at[pl.ds((j + 1) * _W, _W)]], buf1, gs1)
            g0.wait()
            w0 = pltpu.async_copy(buf0, o_hbm.at[pl.ds(r0, _W)], ws0)
            g1.wait()
            w1 = pltpu.async_copy(buf1, o_hbm.at[pl.ds(r0 + _W, _W)], ws1)
            w0.wait()
            w1.wait()

    return k(table2, idx)


def kernel(x, table):
    b, t = x.shape
    v, d = table.shape
    xf = x.reshape(-1).astype(jnp.int32)
    idx2 = (xf[:, None] * _S
            + jnp.arange(_S, dtype=jnp.int32)[None, :]).reshape(-1)
    table2 = table.reshape(v * _S, d // _S)
    out2 = _gather(idx2, table2)
    return out2.reshape(b, t, d)


# speedup vs baseline: 3.5365x; 3.5365x over previous
import jax
import jax.numpy as jnp
from jax import lax
from jax.experimental import pallas as pl
from jax.experimental.pallas import tpu as pltpu
from jax.experimental.pallas import tpu_sc as plsc

_W = 4
_NW = 32


def kernel(x, table):
    b, t = x.shape
    v, d = table.shape
    n = b * t
    k_per_w = n // _NW
    nc = k_per_w // _W
    t_per_w = t // (_NW // b)
    mesh = plsc.VectorSubcoreMesh(core_axis_name="core",
                                  subcore_axis_name="subcore")

    xp = jnp.zeros((n // _W, 8), jnp.int32)
    xp = xp.at[:, :_W].set(x.reshape(n // _W, _W).astype(jnp.int32))
    xp = xp.reshape(-1)

    @pl.kernel(
        out_type=jax.ShapeDtypeStruct((b, t, d), table.dtype),
        mesh=mesh,
        scratch_types=[
            pltpu.VMEM((nc * 8,), jnp.int32),
            pltpu.VMEM((_W, d), jnp.float32),
            pltpu.VMEM((_W, d), jnp.float32),
            pltpu.SemaphoreType.DMA,
            pltpu.SemaphoreType.DMA,
            pltpu.SemaphoreType.DMA,
            pltpu.SemaphoreType.DMA,
            pltpu.SemaphoreType.DMA,
        ],
    )
    def k(table_hbm, i_hbm, o_hbm, idx_v, buf0, buf1, gs0, gs1, ws0, ws1,
          isem):
        cid = lax.axis_index("core")
        sid = lax.axis_index("subcore")
        wid = sid * 2 + cid
        bq = wid // (_NW // b)
        t0 = (wid % (_NW // b)) * t_per_w
        pltpu.async_copy(i_hbm.at[pl.ds(wid * nc * 8, nc * 8)], idx_v,
                         isem).wait()

        @pl.loop(0, nc, step=2)
        def _(j):
            g0 = pltpu.async_copy(
                table_hbm.at[idx_v.at[pl.ds(j * 8, _W)]], buf0, gs0)
            g1 = pltpu.async_copy(
                table_hbm.at[idx_v.at[pl.ds((j + 1) * 8, _W)]], buf1, gs1)
            g0.wait()
            w0 = pltpu.async_copy(
                buf0, o_hbm.at[bq, pl.ds(t0 + j * _W, _W)], ws0)
            g1.wait()
            w1 = pltpu.async_copy(
                buf1, o_hbm.at[bq, pl.ds(t0 + (j + 1) * _W, _W)], ws1)
            w0.wait()
            w1.wait()

    return k(table, xp)


# speedup vs baseline: 3.6648x; 1.0363x over previous
import jax
import jax.numpy as jnp
from jax import lax
from jax.experimental import pallas as pl
from jax.experimental.pallas import tpu as pltpu
from jax.experimental.pallas import tpu_sc as plsc

_W = 4
_NW = 32


def kernel(x, table):
    b, t = x.shape
    v, d = table.shape
    n = b * t
    k_per_w = n // _NW
    nc = k_per_w // _W
    t_per_w = t // (_NW // b)
    mesh = plsc.VectorSubcoreMesh(core_axis_name="core",
                                  subcore_axis_name="subcore")

    xp = jnp.zeros((n // _W, 8), jnp.int32)
    xp = xp.at[:, :_W].set(x.reshape(n // _W, _W).astype(jnp.int32))
    xp = xp.reshape(-1)

    nbuf = 3

    @pl.kernel(
        out_type=jax.ShapeDtypeStruct((b, t, d), table.dtype),
        mesh=mesh,
        scratch_types=[
            pltpu.VMEM((nc * 8,), jnp.int32),
            [pltpu.VMEM((_W, d), jnp.float32) for _ in range(nbuf)],
            [pltpu.SemaphoreType.DMA for _ in range(nbuf)],
            [pltpu.SemaphoreType.DMA for _ in range(nbuf)],
            pltpu.SemaphoreType.DMA,
        ],
    )
    def k(table_hbm, i_hbm, o_hbm, idx_v, bufs, gs, ws, isem):
        cid = lax.axis_index("core")
        sid = lax.axis_index("subcore")
        wid = sid * 2 + cid
        bq = wid // (_NW // b)
        t0 = (wid % (_NW // b)) * t_per_w
        pltpu.async_copy(i_hbm.at[pl.ds(wid * nc * 8, nc * 8)], idx_v,
                         isem).wait()

        def gather(c, u):
            pltpu.async_copy(
                table_hbm.at[idx_v.at[pl.ds(c * 8, _W)]], bufs[u], gs[u])

        def write(c, u):
            pltpu.async_copy(
                bufs[u], o_hbm.at[bq, pl.ds(t0 + c * _W, _W)], ws[u])

        for u in range(nbuf):
            gather(u, u)

        @pl.loop(0, nc + (-nc) % nbuf, step=nbuf)
        def _(j):
            for u in range(nbuf):
                c = j + u

                @pl.when(c < nc)
                def _(c=c, u=u):
                    pltpu.make_async_copy(
                        table_hbm.at[idx_v.at[pl.ds(c * 8, _W)]], bufs[u],
                        gs[u]).wait()
                    write(c, u)

            for u in range(nbuf):
                c = j + u

                @pl.when(c < nc)
                def _(c=c, u=u):
                    pltpu.make_async_copy(
                        bufs[u], o_hbm.at[bq, pl.ds(t0 + c * _W, _W)],
                        ws[u]).wait()

                    @pl.when(c + nbuf < nc)
                    def _():
                        gather(c + nbuf, u)

    return k(table, xp)
